# baseline (speedup 1.0000x reference)
import jax
import jax.numpy as jnp
from jax.experimental import pallas as pl
from jax.experimental.pallas import tpu as pltpu

_FEAT = 3 * 224 * 224
_SUB = 1176
_LANE = 128


def _fma_body(t_ref, sac_ref, somac_ref, x_ref, n_ref, out_ref, ncopy_ref):
    b = pl.program_id(0)
    ti = t_ref[b]
    a = sac_ref[ti]
    c = somac_ref[ti]
    nv = n_ref[...]
    out_ref[...] = a * x_ref[...] + c * nv
    ncopy_ref[...] = nv


def kernel(x_0, t, noise, sqrt_alphas_cumprod, sqrt_one_minus_alphas_cumprod):
    batch = x_0.shape[0]
    x2 = x_0.reshape(batch, _SUB, _LANE)
    n2 = noise.reshape(batch, _SUB, _LANE)

    grid_spec = pltpu.PrefetchScalarGridSpec(
        num_scalar_prefetch=3,
        grid=(batch,),
        in_specs=[
            pl.BlockSpec((1, _SUB, _LANE), lambda b, t_r, sac_r, somac_r: (b, 0, 0)),
            pl.BlockSpec((1, _SUB, _LANE), lambda b, t_r, sac_r, somac_r: (b, 0, 0)),
        ],
        out_specs=[
            pl.BlockSpec((1, _SUB, _LANE), lambda b, t_r, sac_r, somac_r: (b, 0, 0)),
            pl.BlockSpec((1, _SUB, _LANE), lambda b, t_r, sac_r, somac_r: (b, 0, 0)),
        ],
    )

    noisy, ncopy = pl.pallas_call(
        _fma_body,
        grid_spec=grid_spec,
        out_shape=[
            jax.ShapeDtypeStruct((batch, _SUB, _LANE), jnp.float32),
            jax.ShapeDtypeStruct((batch, _SUB, _LANE), jnp.float32),
        ],
    )(t, sqrt_alphas_cumprod, sqrt_one_minus_alphas_cumprod, x2, n2)

    return noisy.reshape(x_0.shape), ncopy.reshape(x_0.shape)


# speedup vs baseline: 1.0783x; 1.0783x over previous
import jax
import jax.numpy as jnp
from jax.experimental import pallas as pl
from jax.experimental.pallas import tpu as pltpu

_FEAT = 3 * 224 * 224
_SUB = 1176
_LANE = 128
_BG = 8


def _fma_body(t_ref, sac_ref, somac_ref, x_ref, n_ref, out_ref, ncopy_ref):
    g = pl.program_id(0)
    for i in range(_BG):
        ti = t_ref[g * _BG + i]
        a = sac_ref[ti]
        c = somac_ref[ti]
        nv = n_ref[i]
        out_ref[i] = a * x_ref[i] + c * nv
        ncopy_ref[i] = nv


def kernel(x_0, t, noise, sqrt_alphas_cumprod, sqrt_one_minus_alphas_cumprod):
    batch = x_0.shape[0]
    x2 = x_0.reshape(batch, _SUB, _LANE)
    n2 = noise.reshape(batch, _SUB, _LANE)

    grid_spec = pltpu.PrefetchScalarGridSpec(
        num_scalar_prefetch=3,
        grid=(batch // _BG,),
        in_specs=[
            pl.BlockSpec((_BG, _SUB, _LANE), lambda g, t_r, sac_r, somac_r: (g, 0, 0)),
            pl.BlockSpec((_BG, _SUB, _LANE), lambda g, t_r, sac_r, somac_r: (g, 0, 0)),
        ],
        out_specs=[
            pl.BlockSpec((_BG, _SUB, _LANE), lambda g, t_r, sac_r, somac_r: (g, 0, 0)),
            pl.BlockSpec((_BG, _SUB, _LANE), lambda g, t_r, sac_r, somac_r: (g, 0, 0)),
        ],
    )

    noisy, ncopy = pl.pallas_call(
        _fma_body,
        grid_spec=grid_spec,
        out_shape=[
            jax.ShapeDtypeStruct((batch, _SUB, _LANE), jnp.float32),
            jax.ShapeDtypeStruct((batch, _SUB, _LANE), jnp.float32),
        ],
    )(t, sqrt_alphas_cumprod, sqrt_one_minus_alphas_cumprod, x2, n2)

    return noisy.reshape(x_0.shape), ncopy.reshape(x_0.shape)


# speedup vs baseline: 4.8993x; 4.5437x over previous
import jax
import jax.numpy as jnp
from jax.experimental import pallas as pl
from jax.experimental.pallas import tpu as pltpu

_C = 3
_H = 224
_W = 224
_BG = 8


def _fma_body(t_ref, sac_ref, somac_ref, x_ref, n_ref, out_ref, ncopy_ref):
    g = pl.program_id(0)
    for i in range(_BG):
        ti = t_ref[g * _BG + i]
        a = sac_ref[ti]
        c = somac_ref[ti]
        nv = n_ref[i]
        out_ref[i] = a * x_ref[i] + c * nv
        ncopy_ref[i] = nv


def kernel(x_0, t, noise, sqrt_alphas_cumprod, sqrt_one_minus_alphas_cumprod):
    batch = x_0.shape[0]

    grid_spec = pltpu.PrefetchScalarGridSpec(
        num_scalar_prefetch=3,
        grid=(batch // _BG,),
        in_specs=[
            pl.BlockSpec((_BG, _C, _H, _W), lambda g, t_r, sac_r, somac_r: (g, 0, 0, 0)),
            pl.BlockSpec((_BG, _C, _H, _W), lambda g, t_r, sac_r, somac_r: (g, 0, 0, 0)),
        ],
        out_specs=[
            pl.BlockSpec((_BG, _C, _H, _W), lambda g, t_r, sac_r, somac_r: (g, 0, 0, 0)),
            pl.BlockSpec((_BG, _C, _H, _W), lambda g, t_r, sac_r, somac_r: (g, 0, 0, 0)),
        ],
    )

    noisy, ncopy = pl.pallas_call(
        _fma_body,
        grid_spec=grid_spec,
        out_shape=[
            jax.ShapeDtypeStruct(x_0.shape, jnp.float32),
            jax.ShapeDtypeStruct(x_0.shape, jnp.float32),
        ],
    )(t, sqrt_alphas_cumprod, sqrt_one_minus_alphas_cumprod, x_0, noise)

    return noisy, ncopy
